# baseline (speedup 1.0000x reference)
import functools

import jax
import jax.numpy as jnp
from jax import lax
from jax.experimental import pallas as pl
from jax.experimental.pallas import tpu as pltpu
from jax.experimental.pallas import tpu_sc as plsc

NC, NS, L = 2, 16, 16
NW = NC * NS
IN_DIM = 128
CB = 128
GROUPS = CB // L


@functools.lru_cache(maxsize=None)
def _build(n_rel, e_pad):
    sub_per_rel = NW // n_rel
    e_per_sub = e_pad // sub_per_rel
    n_chunks = e_per_sub // CB
    mesh = plsc.VectorSubcoreMesh(core_axis_name="c", subcore_axis_name="s")

    @functools.partial(
        pl.kernel,
        out_type=jax.ShapeDtypeStruct((n_rel, e_pad), jnp.float32),
        mesh=mesh,
        compiler_params=pltpu.CompilerParams(needs_layout_passes=False),
        scratch_types=[
            pltpu.VMEM((CB,), jnp.int32),
            pltpu.VMEM((CB,), jnp.int32),
            pltpu.VMEM((CB, IN_DIM), jnp.float32),
            pltpu.VMEM((CB, IN_DIM), jnp.float32),
            pltpu.VMEM((IN_DIM,), jnp.float32),
            pltpu.VMEM((CB,), jnp.float32),
            pltpu.SemaphoreType.DMA,
            pltpu.SemaphoreType.DMA,
        ],
    )
    def decode(z_hbm, src_hbm, dst_hbm, w_hbm, out_hbm,
               si_v, di_v, sr_v, dr_v, w_v, o_v, sem_s, sem_d):
        wid = lax.axis_index("s") * NC + lax.axis_index("c")
        r = wid // sub_per_rel
        sub = wid % sub_per_rel
        sub_base = sub * e_per_sub

        pltpu.sync_copy(w_hbm.at[r], w_v)
        lane = lax.iota(jnp.int32, L)

        def chunk_body(c, _):
            eb = sub_base + c * CB
            pltpu.sync_copy(src_hbm.at[r, pl.ds(eb, CB)], si_v)
            pltpu.sync_copy(dst_hbm.at[r, pl.ds(eb, CB)], di_v)
            cp_s = pltpu.async_copy(z_hbm.at[si_v], sr_v, sem_s)
            cp_d = pltpu.async_copy(z_hbm.at[di_v], dr_v, sem_d)
            cp_s.wait()
            cp_d.wait()

            def d_body(d, accs):
                dcol = jnp.full((L,), d, jnp.int32)
                wv = plsc.load_gather(w_v, [dcol])
                out = []
                for g in range(GROUPS):
                    rows = lane + (g * L)
                    s = plsc.load_gather(sr_v, [rows, dcol])
                    t = plsc.load_gather(dr_v, [rows, dcol])
                    out.append(accs[g] + s * t * wv)
                return tuple(out)

            accs = lax.fori_loop(
                0, IN_DIM, d_body,
                tuple(jnp.zeros((L,), jnp.float32) for _ in range(GROUPS)))
            for g in range(GROUPS):
                o_v[pl.ds(g * L, L)] = 1.0 / (1.0 + jnp.exp(-accs[g]))
            pltpu.sync_copy(o_v, out_hbm.at[r, pl.ds(eb, CB)])
            return 0

        lax.fori_loop(0, n_chunks, chunk_body, 0)

    return decode


def kernel(z, edge_index, weight):
    n_rel, _, e = edge_index.shape
    sub_per_rel = NW // n_rel
    quantum = sub_per_rel * CB
    e_pad = ((e + quantum - 1) // quantum) * quantum
    idx = edge_index.astype(jnp.int32)
    pad = ((0, 0), (0, e_pad - e))
    src = jnp.pad(idx[:, 0, :], pad)
    dst = jnp.pad(idx[:, 1, :], pad)
    out = _build(n_rel, e_pad)(z, src, dst, weight.astype(jnp.float32))
    return out[:, :e]


# speedup vs baseline: 1.0704x; 1.0704x over previous
import functools

import jax
import jax.numpy as jnp
from jax import lax
from jax.experimental import pallas as pl
from jax.experimental.pallas import tpu as pltpu
from jax.experimental.pallas import tpu_sc as plsc

NC, NS, L = 2, 16, 16
NW = NC * NS
IN_DIM = 128
CB = 128
GROUPS = CB // L


@functools.lru_cache(maxsize=None)
def _build(n_rel, e_pad):
    sub_per_rel = NW // n_rel
    e_per_sub = e_pad // sub_per_rel
    n_chunks = e_per_sub // CB
    assert n_chunks % 2 == 0
    mesh = plsc.VectorSubcoreMesh(core_axis_name="c", subcore_axis_name="s")

    @functools.partial(
        pl.kernel,
        out_type=jax.ShapeDtypeStruct((n_rel * e_pad,), jnp.float32),
        mesh=mesh,
        compiler_params=pltpu.CompilerParams(needs_layout_passes=False),
        scratch_types=[
            pltpu.VMEM((e_per_sub + CB,), jnp.int32),
            pltpu.VMEM((e_per_sub + CB,), jnp.int32),
            pltpu.VMEM((CB, IN_DIM), jnp.float32),
            pltpu.VMEM((CB, IN_DIM), jnp.float32),
            pltpu.VMEM((CB, IN_DIM), jnp.float32),
            pltpu.VMEM((CB, IN_DIM), jnp.float32),
            pltpu.VMEM((IN_DIM,), jnp.float32),
            pltpu.VMEM((e_per_sub,), jnp.float32),
            pltpu.SemaphoreType.DMA,
            pltpu.SemaphoreType.DMA,
        ],
    )
    def decode(z_hbm, src_hbm, dst_hbm, w_hbm, out_hbm,
               si, di, sr0, dr0, sr1, dr1, w_v, o_all, sem0, sem1):
        wid = lax.axis_index("s") * NC + lax.axis_index("c")
        r = wid // sub_per_rel
        base = wid * e_per_sub

        pltpu.sync_copy(w_hbm.at[r], w_v)
        pltpu.sync_copy(src_hbm.at[pl.ds(base, e_per_sub + CB)], si)
        pltpu.sync_copy(dst_hbm.at[pl.ds(base, e_per_sub + CB)], di)

        bufs = ((sr0, dr0, sem0), (sr1, dr1, sem1))
        lane = lax.iota(jnp.int32, L)

        def start_gather(c, p):
            sr, dr, sem = bufs[p]
            pltpu.async_copy(z_hbm.at[si.at[pl.ds(c * CB, CB)]], sr, sem)
            pltpu.async_copy(z_hbm.at[di.at[pl.ds(c * CB, CB)]], dr, sem)

        def wait_gather(c, p):
            sr, dr, sem = bufs[p]
            pltpu.make_async_copy(z_hbm.at[si.at[pl.ds(c * CB, CB)]], sr, sem).wait()
            pltpu.make_async_copy(z_hbm.at[di.at[pl.ds(c * CB, CB)]], dr, sem).wait()

        def compute(c, p):
            sr, dr, _ = bufs[p]

            def d_body(d, accs):
                dcol = jnp.full((L,), d, jnp.int32)
                wv = plsc.load_gather(w_v, [dcol])
                out = []
                for g in range(GROUPS):
                    rows = lane + (g * L)
                    s = plsc.load_gather(sr, [rows, dcol])
                    t = plsc.load_gather(dr, [rows, dcol])
                    out.append(accs[g] + s * t * wv)
                return tuple(out)

            accs = lax.fori_loop(
                0, IN_DIM, d_body,
                tuple(jnp.zeros((L,), jnp.float32) for _ in range(GROUPS)))
            for g in range(GROUPS):
                o_all[pl.ds(c * CB + g * L, L)] = 1.0 / (1.0 + jnp.exp(-accs[g]))

        start_gather(0, 0)

        def pair_body(i, _):
            c0 = i * 2
            for b in range(2):
                c = c0 + b
                start_gather(c + 1, 1 - b)
                wait_gather(c, b)
                compute(c, b)
            return 0

        lax.fori_loop(0, n_chunks // 2, pair_body, 0)
        wait_gather(n_chunks, 0)
        pltpu.sync_copy(o_all, out_hbm.at[pl.ds(base, e_per_sub)])

    return decode


def kernel(z, edge_index, weight):
    n_rel, _, e = edge_index.shape
    sub_per_rel = NW // n_rel
    quantum = sub_per_rel * CB * 2
    e_pad = ((e + quantum - 1) // quantum) * quantum
    idx = edge_index.astype(jnp.int32)
    src = jnp.pad(idx[:, 0, :], ((0, 0), (0, e_pad - e))).reshape(-1)
    dst = jnp.pad(idx[:, 1, :], ((0, 0), (0, e_pad - e))).reshape(-1)
    src = jnp.pad(src, (0, CB))
    dst = jnp.pad(dst, (0, CB))
    out = _build(n_rel, e_pad)(z, src, dst, weight.astype(jnp.float32))
    return out.reshape(n_rel, e_pad)[:, :e]


# speedup vs baseline: 4.0834x; 3.8148x over previous
import functools

import jax
import jax.numpy as jnp
from jax import lax
from jax.experimental import pallas as pl
from jax.experimental.pallas import tpu as pltpu
from jax.experimental.pallas import tpu_sc as plsc

NC, NS, L = 2, 16, 16
NW = NC * NS
IN_DIM = 128
CB = 128
GROUPS = CB // L


@functools.lru_cache(maxsize=None)
def _build(n_rel, e_pad):
    sub_per_rel = NW // n_rel
    e_per_sub = e_pad // sub_per_rel
    n_chunks = e_per_sub // CB
    assert n_chunks % 2 == 0
    mesh = plsc.VectorSubcoreMesh(core_axis_name="c", subcore_axis_name="s")

    @functools.partial(
        pl.kernel,
        out_type=jax.ShapeDtypeStruct((n_rel * e_pad,), jnp.float32),
        mesh=mesh,
        compiler_params=pltpu.CompilerParams(needs_layout_passes=False),
        scratch_types=[
            pltpu.VMEM((e_per_sub + CB,), jnp.int32),
            pltpu.VMEM((e_per_sub + CB,), jnp.int32),
            pltpu.VMEM((CB, IN_DIM), jnp.float32),
            pltpu.VMEM((CB, IN_DIM), jnp.float32),
            pltpu.VMEM((CB, IN_DIM), jnp.float32),
            pltpu.VMEM((CB, IN_DIM), jnp.float32),
            pltpu.VMEM((IN_DIM,), jnp.float32),
            pltpu.VMEM((e_per_sub,), jnp.float32),
            pltpu.SemaphoreType.DMA,
            pltpu.SemaphoreType.DMA,
        ],
    )
    def decode(z_hbm, src_hbm, dst_hbm, w_hbm, out_hbm,
               si, di, sr0, dr0, sr1, dr1, w_v, o_all, sem0, sem1):
        wid = lax.axis_index("s") * NC + lax.axis_index("c")
        r = wid // sub_per_rel
        base = wid * e_per_sub

        pltpu.sync_copy(w_hbm.at[r], w_v)
        pltpu.sync_copy(src_hbm.at[pl.ds(base, e_per_sub + CB)], si)
        pltpu.sync_copy(dst_hbm.at[pl.ds(base, e_per_sub + CB)], di)

        bufs = ((sr0, dr0, sem0), (sr1, dr1, sem1))
        lane = lax.iota(jnp.int32, L)

        def start_gather(c, p):
            sr, dr, sem = bufs[p]
            pltpu.async_copy(z_hbm.at[si.at[pl.ds(c * CB, CB)]], sr, sem)
            pltpu.async_copy(z_hbm.at[di.at[pl.ds(c * CB, CB)]], dr, sem)

        def wait_gather(c, p):
            sr, dr, sem = bufs[p]
            pltpu.make_async_copy(z_hbm.at[si.at[pl.ds(c * CB, CB)]], sr, sem).wait()
            pltpu.make_async_copy(z_hbm.at[di.at[pl.ds(c * CB, CB)]], dr, sem).wait()

        n_k = IN_DIM // L
        wks = [w_v[pl.ds(k * L, L)] for k in range(n_k)]

        def compute(c, p):
            sr, dr, _ = bufs[p]

            def g_body(g, _):
                ebase = g * L

                def q_body(q, ovec):
                    for j in range(4):
                        e = q * 4 + j
                        acc = None
                        for k in range(n_k):
                            t = (sr[ebase + e, pl.ds(k * L, L)]
                                 * dr[ebase + e, pl.ds(k * L, L)]) * wks[k]
                            acc = t if acc is None else acc + t
                        ovec = jnp.where(lane == e, jnp.sum(acc), ovec)
                    return ovec

                ovec = lax.fori_loop(0, 4, q_body, jnp.zeros((L,), jnp.float32))
                o_all[pl.ds(c * CB + g * L, L)] = 1.0 / (1.0 + jnp.exp(-ovec))
                return 0

            lax.fori_loop(0, GROUPS, g_body, 0)

        start_gather(0, 0)

        def pair_body(i, _):
            c0 = i * 2
            for b in range(2):
                c = c0 + b
                start_gather(c + 1, 1 - b)
                wait_gather(c, b)
                compute(c, b)
            return 0

        lax.fori_loop(0, n_chunks // 2, pair_body, 0)
        wait_gather(n_chunks, 0)
        pltpu.sync_copy(o_all, out_hbm.at[pl.ds(base, e_per_sub)])

    return decode


def kernel(z, edge_index, weight):
    n_rel, _, e = edge_index.shape
    sub_per_rel = NW // n_rel
    quantum = sub_per_rel * CB * 2
    e_pad = ((e + quantum - 1) // quantum) * quantum
    idx = edge_index.astype(jnp.int32)
    src = jnp.pad(idx[:, 0, :], ((0, 0), (0, e_pad - e))).reshape(-1)
    dst = jnp.pad(idx[:, 1, :], ((0, 0), (0, e_pad - e))).reshape(-1)
    src = jnp.pad(src, (0, CB))
    dst = jnp.pad(dst, (0, CB))
    out = _build(n_rel, e_pad)(z, src, dst, weight.astype(jnp.float32))
    return out.reshape(n_rel, e_pad)[:, :e]


# speedup vs baseline: 4.5043x; 1.1031x over previous
import functools

import jax
import jax.numpy as jnp
from jax import lax
from jax.experimental import pallas as pl
from jax.experimental.pallas import tpu as pltpu
from jax.experimental.pallas import tpu_sc as plsc

NC, NS, L = 2, 16, 16
NW = NC * NS
IN_DIM = 128
PK = IN_DIM // 2
CB = 128
GROUPS = CB // L


@functools.lru_cache(maxsize=None)
def _build(n_rel, e_pad):
    sub_per_rel = NW // n_rel
    e_per_sub = e_pad // sub_per_rel
    n_chunks = e_per_sub // CB
    assert n_chunks % 2 == 0
    mesh = plsc.VectorSubcoreMesh(core_axis_name="c", subcore_axis_name="s")

    @functools.partial(
        pl.kernel,
        out_type=jax.ShapeDtypeStruct((n_rel * e_pad,), jnp.float32),
        mesh=mesh,
        compiler_params=pltpu.CompilerParams(
            needs_layout_passes=False, use_tc_tiling_on_sc=False),
        scratch_types=[
            pltpu.VMEM((e_per_sub + CB,), jnp.int32),
            pltpu.VMEM((e_per_sub + CB,), jnp.int32),
            pltpu.VMEM((CB, PK), jnp.float32),
            pltpu.VMEM((CB, PK), jnp.float32),
            pltpu.VMEM((CB, PK), jnp.float32),
            pltpu.VMEM((CB, PK), jnp.float32),
            pltpu.VMEM((PK,), jnp.float32),
            pltpu.VMEM((e_per_sub,), jnp.float32),
            pltpu.SemaphoreType.DMA,
            pltpu.SemaphoreType.DMA,
        ],
    )
    def decode(z_hbm, src_hbm, dst_hbm, w_hbm, out_hbm,
               si, di, sr0, dr0, sr1, dr1, w_v, o_all, sem0, sem1):
        wid = lax.axis_index("s") * NC + lax.axis_index("c")
        r = wid // sub_per_rel
        base = wid * e_per_sub

        pltpu.sync_copy(w_hbm.at[r], w_v)
        pltpu.sync_copy(src_hbm.at[pl.ds(base, e_per_sub + CB)], si)
        pltpu.sync_copy(dst_hbm.at[pl.ds(base, e_per_sub + CB)], di)

        bufs = ((sr0, dr0, sem0), (sr1, dr1, sem1))
        lane = lax.iota(jnp.int32, L)

        def start_gather(c, p):
            sr, dr, sem = bufs[p]
            pltpu.async_copy(z_hbm.at[si.at[pl.ds(c * CB, CB)]], sr, sem)
            pltpu.async_copy(z_hbm.at[di.at[pl.ds(c * CB, CB)]], dr, sem)

        def wait_gather(c, p):
            sr, dr, sem = bufs[p]
            pltpu.make_async_copy(z_hbm.at[si.at[pl.ds(c * CB, CB)]], sr, sem).wait()
            pltpu.make_async_copy(z_hbm.at[di.at[pl.ds(c * CB, CB)]], dr, sem).wait()

        n_k = PK // L
        wab = []
        for k in range(n_k):
            wbf = plsc.bitcast(w_v[pl.ds(k * L, L)], jnp.bfloat16)
            wab.append(plsc.unpack(wbf, format=plsc.PackFormat.INTERLEAVED))

        def compute(c, p):
            sr, dr, _ = bufs[p]

            def g_body(g, _):
                ebase = g * L

                def q_body(q, ovec):
                    for j in range(4):
                        e = q * 4 + j
                        acc = None
                        for k in range(n_k):
                            sb = plsc.bitcast(sr[ebase + e, pl.ds(k * L, L)],
                                              jnp.bfloat16)
                            db = plsc.bitcast(dr[ebase + e, pl.ds(k * L, L)],
                                              jnp.bfloat16)
                            pa, pb = plsc.unpack(sb * db,
                                                 format=plsc.PackFormat.INTERLEAVED)
                            wa, wb = wab[k]
                            t = pa * wa + pb * wb
                            acc = t if acc is None else acc + t
                        ovec = jnp.where(lane == e, jnp.sum(acc), ovec)
                    return ovec

                ovec = lax.fori_loop(0, 4, q_body, jnp.zeros((L,), jnp.float32))
                o_all[pl.ds(c * CB + g * L, L)] = 1.0 / (1.0 + jnp.exp(-ovec))
                return 0

            lax.fori_loop(0, GROUPS, g_body, 0)

        start_gather(0, 0)

        def pair_body(i, _):
            c0 = i * 2
            for b in range(2):
                c = c0 + b
                start_gather(c + 1, 1 - b)
                wait_gather(c, b)
                compute(c, b)
            return 0

        lax.fori_loop(0, n_chunks // 2, pair_body, 0)
        wait_gather(n_chunks, 0)
        pltpu.sync_copy(o_all, out_hbm.at[pl.ds(base, e_per_sub)])

    return decode


def kernel(z, edge_index, weight):
    n_rel, _, e = edge_index.shape
    sub_per_rel = NW // n_rel
    quantum = sub_per_rel * CB * 2
    e_pad = ((e + quantum - 1) // quantum) * quantum
    idx = edge_index.astype(jnp.int32)
    src = jnp.pad(idx[:, 0, :], ((0, 0), (0, e_pad - e))).reshape(-1)
    dst = jnp.pad(idx[:, 1, :], ((0, 0), (0, e_pad - e))).reshape(-1)
    src = jnp.pad(src, (0, CB))
    dst = jnp.pad(dst, (0, CB))
    z_pk = lax.bitcast_convert_type(
        z.astype(jnp.bfloat16).reshape(z.shape[0], PK, 2), jnp.float32)
    w_pk = lax.bitcast_convert_type(
        weight.astype(jnp.bfloat16).reshape(weight.shape[0], PK, 2), jnp.float32)
    out = _build(n_rel, e_pad)(z_pk, src, dst, w_pk)
    return out.reshape(n_rel, e_pad)[:, :e]
